# baseline (speedup 1.0000x reference)
import functools

import jax
import jax.numpy as jnp
from jax import lax
from jax.experimental import pallas as pl
from jax.experimental.pallas import tpu as pltpu
from jax.experimental.pallas import tpu_sc as plsc

N = 8 * 512 * 512
NW = 32
ROWS_W = 512 // 4
RCHUNK = 32
NCHUNK = ROWS_W // RCHUNK
KB = 1024
HR = 2 * KB // 128
L = 16
EPS = 1e-10
GRID = 4


@functools.cache
def _build_sc_hist():
    mesh = plsc.VectorSubcoreMesh(core_axis_name="c", subcore_axis_name="s")
    return functools.partial(
        pl.kernel,
        mesh=mesh,
        out_type=jax.ShapeDtypeStruct((NW * L, HR, 128), jnp.float32),
        scratch_types=[
            pltpu.VMEM((2 * RCHUNK, 512), jnp.float32),
            pltpu.VMEM((2 * RCHUNK, 512), jnp.int32),
            pltpu.VMEM((L, HR, 128), jnp.float32),
            pltpu.SemaphoreType.DMA((2,)),
        ],
        compiler_params=pltpu.CompilerParams(needs_layout_passes=False),
    )(_sc_hist_body)


def _sc_hist_body(pred_hbm, tgt_hbm, out_cnt, pbuf, tbuf, hcnt, sems):
    wid = lax.axis_index("c") * 16 + lax.axis_index("s")
    zeros = jnp.zeros((L,), jnp.float32)
    ones = jnp.ones((L,), jnp.float32)
    lane = lax.iota(jnp.int32, L)

    @plsc.parallel_loop(0, L * HR * 8, unroll=16)
    def _zero(i):
        hcnt[i >> 7, (i >> 3) & (HR - 1), pl.ds((i & 7) * L, L)] = zeros

    img = wid >> 2
    row0 = (wid & 3) * ROWS_W

    def _start(rbase, par):
        roff = par * RCHUNK
        pltpu.async_copy(
            pred_hbm.at[img, pl.ds(rbase, RCHUNK), :],
            pbuf.at[pl.ds(roff, RCHUNK), :], sems.at[par])
        pltpu.async_copy(
            tgt_hbm.at[img, pl.ds(rbase, RCHUNK), :],
            tbuf.at[pl.ds(roff, RCHUNK), :], sems.at[par])

    def _wait(rbase, par):
        roff = par * RCHUNK
        pltpu.make_async_copy(
            pred_hbm.at[img, pl.ds(rbase, RCHUNK), :],
            pbuf.at[pl.ds(roff, RCHUNK), :], sems.at[par]).wait()
        pltpu.make_async_copy(
            tgt_hbm.at[img, pl.ds(rbase, RCHUNK), :],
            tbuf.at[pl.ds(roff, RCHUNK), :], sems.at[par]).wait()

    _start(row0, 0)

    def _chunk(ci, carry):
        par = ci & 1
        rbase = row0 + ci * RCHUNK
        _wait(rbase, par)

        @pl.when(ci + 1 < NCHUNK)
        def _prefetch():
            _start(rbase + RCHUNK, 1 - par)

        roff = par * RCHUNK

        @plsc.parallel_loop(0, RCHUNK * 512 // L, unroll=16)
        def _step(i):
            r = roff + (i >> 5)
            c = (i & 31) * L
            vp = pbuf[r, pl.ds(c, L)]
            vt = tbuf[r, pl.ds(c, L)]
            b = jnp.minimum((vp * float(KB)).astype(jnp.int32), KB - 1)
            b = jnp.maximum(b, 0)
            key = ((KB - 1) - b) + (vt << 10)
            plsc.addupdate_scatter(hcnt, [lane, key >> 7, key & 127], ones)

        return carry

    lax.fori_loop(0, NCHUNK, _chunk, 0)
    pltpu.sync_copy(hcnt, out_cnt.at[pl.ds(wid * L, L)])


def _combine_body(cnt_ref, out_ref, acc_vmem):
    g = pl.program_id(0)
    csum = jnp.sum(cnt_ref[...], axis=0)

    @pl.when(g == 0)
    def _init():
        acc_vmem[...] = csum

    @pl.when(g > 0)
    def _acc():
        acc_vmem[...] += csum

    @pl.when(g == GRID - 1)
    def _final():
        nf = float(N)
        kbf = float(KB)
        c16 = acc_vmem[...]
        cntn = c16[0:8, :]
        cntp = c16[8:16, :]
        n_neg = jnp.sum(cntn)
        gts = nf - n_neg

        bidx = (lax.broadcasted_iota(jnp.int32, (8, 128), 0) * 128
                + lax.broadcasted_iota(jnp.int32, (8, 128), 1)
                ).astype(jnp.float32)
        pmid = (kbf - 0.5 - bidx) / kbf
        s_pos = jnp.sum(cntp * (1.0 - pmid))
        term1 = s_pos / (nf + EPS)

        iu0 = lax.broadcasted_iota(jnp.int32, (128, 128), 0)
        iu1 = lax.broadcasted_iota(jnp.int32, (128, 128), 1)
        upper = (iu0 <= iu1).astype(jnp.float32)
        im0 = lax.broadcasted_iota(jnp.int32, (8, 8), 0)
        im1 = lax.broadcasted_iota(jnp.int32, (8, 8), 1)
        strict_lower = (im0 > im1).astype(jnp.float32)
        ones128 = jnp.ones((128, 128), jnp.float32)
        incl = jnp.dot(cntn, upper, preferred_element_type=jnp.float32)
        rowtot_b = jnp.dot(cntn, ones128, preferred_element_type=jnp.float32)
        excl_rows = jnp.dot(strict_lower, rowtot_b,
                            preferred_element_type=jnp.float32)
        j_excl = excl_rows + incl - cntn

        a = gts + j_excl + EPS
        term2 = jnp.sum(gts * (cntn * (1.0 + pmid)) / (a * (a + cntn)))

        emax = jnp.max(jnp.where(cntn > 0.0, 1.0 + (kbf - bidx) / kbf, -1.0))
        out_ref[0, 0] = term1 + term2 + jnp.where(gts == 0.0, emax, 0.0)


_combine = pl.pallas_call(
    _combine_body,
    grid=(GRID,),
    in_specs=[
        pl.BlockSpec((NW * L // GRID, HR, 128), lambda i: (i, 0, 0)),
    ],
    out_shape=jax.ShapeDtypeStruct((1, 1), jnp.float32),
    out_specs=pl.BlockSpec(
        (1, 1), lambda i: (0, 0), memory_space=pltpu.SMEM),
    scratch_shapes=[
        pltpu.VMEM((HR, 128), jnp.float32),
    ],
)


def kernel(pred, target):
    cnt = _build_sc_hist()(pred, target)
    loss = _combine(cnt)
    return loss[0, 0]


# speedup vs baseline: 1.0315x; 1.0315x over previous
import functools

import jax
import jax.numpy as jnp
from jax import lax
from jax.experimental import pallas as pl
from jax.experimental.pallas import tpu as pltpu
from jax.experimental.pallas import tpu_sc as plsc

N = 8 * 512 * 512
NW = 32
ROWS_W = 512 // 4
RCHUNK = 32
NCHUNK = ROWS_W // RCHUNK
KB = 1024
HR = 2 * KB // 128
L = 16
EPS = 1e-10
GRID = 2


@functools.cache
def _build_sc_hist():
    mesh = plsc.VectorSubcoreMesh(core_axis_name="c", subcore_axis_name="s")
    return functools.partial(
        pl.kernel,
        mesh=mesh,
        out_type=jax.ShapeDtypeStruct((NW * L, HR, 128), jnp.float32),
        scratch_types=[
            pltpu.VMEM((2 * RCHUNK, 512), jnp.float32),
            pltpu.VMEM((2 * RCHUNK, 512), jnp.int32),
            pltpu.VMEM((L, HR, 128), jnp.float32),
            pltpu.SemaphoreType.DMA((2,)),
        ],
        compiler_params=pltpu.CompilerParams(needs_layout_passes=False),
    )(_sc_hist_body)


def _sc_hist_body(pred_hbm, tgt_hbm, out_cnt, pbuf, tbuf, hcnt, sems):
    wid = lax.axis_index("c") * 16 + lax.axis_index("s")
    zeros = jnp.zeros((L,), jnp.float32)
    ones = jnp.ones((L,), jnp.float32)
    lane = lax.iota(jnp.int32, L)

    @plsc.parallel_loop(0, L * HR * 8, unroll=8)
    def _zero(i):
        hcnt[i >> 7, (i >> 3) & (HR - 1), pl.ds((i & 7) * L, L)] = zeros

    img = wid >> 2
    row0 = (wid & 3) * ROWS_W

    def _start(rbase, par):
        roff = par * RCHUNK
        pltpu.async_copy(
            pred_hbm.at[img, pl.ds(rbase, RCHUNK), :],
            pbuf.at[pl.ds(roff, RCHUNK), :], sems.at[par])
        pltpu.async_copy(
            tgt_hbm.at[img, pl.ds(rbase, RCHUNK), :],
            tbuf.at[pl.ds(roff, RCHUNK), :], sems.at[par])

    def _wait(rbase, par):
        roff = par * RCHUNK
        pltpu.make_async_copy(
            pred_hbm.at[img, pl.ds(rbase, RCHUNK), :],
            pbuf.at[pl.ds(roff, RCHUNK), :], sems.at[par]).wait()
        pltpu.make_async_copy(
            tgt_hbm.at[img, pl.ds(rbase, RCHUNK), :],
            tbuf.at[pl.ds(roff, RCHUNK), :], sems.at[par]).wait()

    _start(row0, 0)

    def _chunk(ci, carry):
        par = ci & 1
        rbase = row0 + ci * RCHUNK
        _wait(rbase, par)

        @pl.when(ci + 1 < NCHUNK)
        def _prefetch():
            _start(rbase + RCHUNK, 1 - par)

        roff = par * RCHUNK

        @plsc.parallel_loop(0, RCHUNK * 512 // L, unroll=8)
        def _step(i):
            r = roff + (i >> 5)
            c = (i & 31) * L
            vp = pbuf[r, pl.ds(c, L)]
            vt = tbuf[r, pl.ds(c, L)]
            b = jnp.minimum((vp * float(KB)).astype(jnp.int32), KB - 1)
            b = jnp.maximum(b, 0)
            key = ((KB - 1) - b) + (vt << 10)
            plsc.addupdate_scatter(hcnt, [lane, key >> 7, key & 127], ones)

        return carry

    lax.fori_loop(0, NCHUNK, _chunk, 0)
    pltpu.sync_copy(hcnt, out_cnt.at[pl.ds(wid * L, L)])


def _combine_body(cnt_ref, out_ref, acc_vmem):
    g = pl.program_id(0)
    csum = jnp.sum(cnt_ref[...], axis=0)

    @pl.when(g == 0)
    def _init():
        acc_vmem[...] = csum

    @pl.when(g > 0)
    def _acc():
        acc_vmem[...] += csum

    @pl.when(g == GRID - 1)
    def _final():
        nf = float(N)
        kbf = float(KB)
        c16 = acc_vmem[...]
        cntn = c16[0:8, :]
        cntp = c16[8:16, :]
        n_neg = jnp.sum(cntn)
        gts = nf - n_neg

        bidx = (lax.broadcasted_iota(jnp.int32, (8, 128), 0) * 128
                + lax.broadcasted_iota(jnp.int32, (8, 128), 1)
                ).astype(jnp.float32)
        pmid = (kbf - 0.5 - bidx) / kbf
        s_pos = jnp.sum(cntp * (1.0 - pmid))
        term1 = s_pos / (nf + EPS)

        iu0 = lax.broadcasted_iota(jnp.int32, (128, 128), 0)
        iu1 = lax.broadcasted_iota(jnp.int32, (128, 128), 1)
        upper = (iu0 <= iu1).astype(jnp.float32)
        im0 = lax.broadcasted_iota(jnp.int32, (8, 8), 0)
        im1 = lax.broadcasted_iota(jnp.int32, (8, 8), 1)
        strict_lower = (im0 > im1).astype(jnp.float32)
        ones128 = jnp.ones((128, 128), jnp.float32)
        incl = jnp.dot(cntn, upper, preferred_element_type=jnp.float32)
        rowtot_b = jnp.dot(cntn, ones128, preferred_element_type=jnp.float32)
        excl_rows = jnp.dot(strict_lower, rowtot_b,
                            preferred_element_type=jnp.float32)
        j_excl = excl_rows + incl - cntn

        a = gts + j_excl + EPS
        term2 = jnp.sum(gts * (cntn * (1.0 + pmid)) / (a * (a + cntn)))

        emax = jnp.max(jnp.where(cntn > 0.0, 1.0 + (kbf - bidx) / kbf, -1.0))
        out_ref[0, 0] = term1 + term2 + jnp.where(gts == 0.0, emax, 0.0)


_combine = pl.pallas_call(
    _combine_body,
    grid=(GRID,),
    in_specs=[
        pl.BlockSpec((NW * L // GRID, HR, 128), lambda i: (i, 0, 0)),
    ],
    out_shape=jax.ShapeDtypeStruct((1, 1), jnp.float32),
    out_specs=pl.BlockSpec(
        (1, 1), lambda i: (0, 0), memory_space=pltpu.SMEM),
    scratch_shapes=[
        pltpu.VMEM((HR, 128), jnp.float32),
    ],
)


def kernel(pred, target):
    cnt = _build_sc_hist()(pred, target)
    loss = _combine(cnt)
    return loss[0, 0]
